# Initial kernel scaffold; baseline (speedup 1.0000x reference)
import jax
import jax.numpy as jnp
from jax.experimental import pallas as pl


def kernel(x, edge_index, edge_attr, W1, b1, W2, b2, Wg, bg, W3, b3, W4, b4):
    raise NotImplementedError("write your pallas kernel here")


# speedup vs baseline: 11.2558x; 11.2558x over previous
import functools

import jax
import jax.numpy as jnp
from jax import lax
from jax.experimental import pallas as pl
from jax.experimental.pallas import tpu as pltpu
from jax.experimental.pallas import tpu_sc as plsc

N, E, D, DE, H = 10000, 320000, 128, 16, 64
NP = 10240
NC, NS = 2, 16
NW = NC * NS
C = 80
EW = E // NW
NCH = EW // C
RWP = NP // NS

_mesh = plsc.VectorSubcoreMesh(core_axis_name="c", subcore_axis_name="s")


@functools.partial(
    pl.kernel,
    out_type=jax.ShapeDtypeStruct((NC, NP), jnp.float32),
    mesh=_mesh,
    scratch_types=[
        pltpu.VMEM((NCH, C), jnp.int32),
        pltpu.VMEM((C,), jnp.float32),
        pltpu.VMEM((RWP,), jnp.float32),
        pltpu.VMEM_SHARED((NP,), jnp.float32),
    ],
    compiler_params=pltpu.CompilerParams(use_tc_tiling_on_sc=False),
)
def _deg_kernel(dst2, deg_hbm, idx_v, ones_v, zbuf_v, deg_s):
    cid = lax.axis_index("c")
    sid = lax.axis_index("s")
    wid = sid * NC + cid

    def z16(i, _):
        zbuf_v[pl.ds(i * 16, 16)] = jnp.zeros((16,), jnp.float32)
        return 0

    lax.fori_loop(0, RWP // 16, z16, 0)
    for k in range(C // 16):
        ones_v[pl.ds(k * 16, 16)] = jnp.ones((16,), jnp.float32)
    pltpu.sync_copy(zbuf_v, deg_s.at[pl.ds(sid * RWP, RWP)])
    pltpu.sync_copy(dst2.at[wid], idx_v)
    plsc.subcore_barrier()

    def body(j, _):
        pltpu.sync_copy(ones_v, deg_s.at[idx_v.at[j]], add=True)
        return 0

    lax.fori_loop(0, NCH, body, 0)
    plsc.subcore_barrier()
    pltpu.sync_copy(deg_s.at[pl.ds(sid * RWP, RWP)], zbuf_v)
    pltpu.sync_copy(zbuf_v, deg_hbm.at[cid, pl.ds(sid * RWP, RWP)])


@functools.partial(
    pl.kernel,
    out_type=jax.ShapeDtypeStruct((NC, NP, H), jnp.float32),
    mesh=_mesh,
    scratch_types=[
        pltpu.VMEM((NCH, C), jnp.int32),
        pltpu.VMEM((NCH, C), jnp.int32),
        pltpu.VMEM((C, H), jnp.float32),
        pltpu.VMEM((C, H), jnp.float32),
        pltpu.VMEM_SHARED((NP, H), jnp.float32),
        pltpu.SemaphoreType.DMA,
    ],
    compiler_params=pltpu.CompilerParams(use_tc_tiling_on_sc=False),
)
def _scatter_kernel(src2, dst2, u_hbm, s_hbm, sidx_v, didx_v, rows_v, nbuf_v,
                    s_s, sem):
    cid = lax.axis_index("c")
    sid = lax.axis_index("s")
    wid = sid * NC + cid

    def zrow(i, _):
        for k in range(H // 16):
            nbuf_v[i, pl.ds(k * 16, 16)] = jnp.zeros((16,), jnp.float32)
        return 0

    lax.fori_loop(0, C, zrow, 0)
    for t in range(RWP // C):
        pltpu.sync_copy(nbuf_v, s_s.at[pl.ds(sid * RWP + t * C, C)])
    pltpu.sync_copy(src2.at[wid], sidx_v)
    pltpu.sync_copy(dst2.at[wid], didx_v)
    plsc.subcore_barrier()

    def body(j, _):
        pltpu.async_copy(u_hbm.at[sidx_v.at[j]], rows_v, sem).wait()
        pltpu.sync_copy(rows_v, s_s.at[didx_v.at[j]], add=True)
        return 0

    lax.fori_loop(0, NCH, body, 0)
    plsc.subcore_barrier()
    for t in range(RWP // C):
        pltpu.sync_copy(s_s.at[pl.ds(sid * RWP + t * C, C)], nbuf_v)
        pltpu.sync_copy(nbuf_v, s_hbm.at[cid, pl.ds(sid * RWP + t * C, C)])


@functools.partial(
    pl.kernel,
    out_type=jax.ShapeDtypeStruct((E, H), jnp.float32),
    mesh=_mesh,
    scratch_types=[
        pltpu.VMEM((NCH, C), jnp.int32),
        pltpu.VMEM((NCH, C), jnp.int32),
        pltpu.VMEM((C, H), jnp.float32),
        pltpu.VMEM((C, H), jnp.float32),
        pltpu.VMEM_SHARED((NP, H), jnp.float32),
        pltpu.VMEM_SHARED((NP, H), jnp.float32),
        pltpu.SemaphoreType.DMA,
        pltpu.SemaphoreType.DMA,
    ],
    compiler_params=pltpu.CompilerParams(use_tc_tiling_on_sc=False),
)
def _gather_kernel(src2, dst2, a_hbm, b_hbm, g_hbm, sidx_v, didx_v, bufa_v,
                   bufb_v, a_s, b_s, sema, semb):
    cid = lax.axis_index("c")
    sid = lax.axis_index("s")
    wid = sid * NC + cid
    pltpu.sync_copy(src2.at[wid], sidx_v)
    pltpu.sync_copy(dst2.at[wid], didx_v)
    for t in range(RWP // C):
        sl = pl.ds(sid * RWP + t * C, C)
        pltpu.sync_copy(a_hbm.at[sl], bufa_v)
        pltpu.sync_copy(bufa_v, a_s.at[sl])
        pltpu.sync_copy(b_hbm.at[sl], bufb_v)
        pltpu.sync_copy(bufb_v, b_s.at[sl])
    plsc.subcore_barrier()

    def body(j, _):
        ca = pltpu.async_copy(a_s.at[sidx_v.at[j]], bufa_v, sema)
        cb = pltpu.async_copy(b_s.at[didx_v.at[j]], bufb_v, semb)
        ca.wait()
        cb.wait()

        def addrow(i, _):
            for k in range(H // 16):
                sl = pl.ds(k * 16, 16)
                bufa_v[i, sl] = bufa_v[i, sl] + bufb_v[i, sl]
            return 0

        lax.fori_loop(0, C, addrow, 0)
        pltpu.sync_copy(bufa_v, g_hbm.at[pl.ds((wid * NCH + j) * C, C)])
        return 0

    lax.fori_loop(0, NCH, body, 0)


def _node1_body(x_ref, w1_ref, b1_ref, wg_ref, degc_ref, u_ref):
    h = jnp.maximum(
        jnp.dot(x_ref[...], w1_ref[...], preferred_element_type=jnp.float32)
        + b1_ref[...], 0.0)
    hw = jnp.dot(h, wg_ref[...], preferred_element_type=jnp.float32)
    d = degc_ref[...]
    dinv = lax.rsqrt(d[:, 0:1] + d[:, 1:2] + 1.0)
    u_ref[:N] = hw * dinv[:N, :]
    u_ref[N:] = jnp.zeros((NP - N, H), jnp.float32)


def _node2_body(sp_ref, u_ref, degc_ref, bg_ref, w3a_ref, w3b_ref, b3_ref,
                a_ref, b_ref):
    s = sp_ref[0] + sp_ref[1] + u_ref[...]
    d = degc_ref[...]
    dinv = lax.rsqrt(d[:, 0:1] + d[:, 1:2] + 1.0)
    agg = s * dinv + bg_ref[...]
    a_ref[...] = jnp.dot(agg, w3a_ref[...], preferred_element_type=jnp.float32)
    b_ref[...] = jnp.dot(agg, w3b_ref[...],
                         preferred_element_type=jnp.float32) + b3_ref[...]


BLK = 8000


def _edge_body(g_ref, ea_ref, w2_ref, b2_ref, w3c_ref, w4_ref, b4_ref, o_ref):
    ef = jnp.maximum(
        jnp.dot(ea_ref[...], w2_ref[...], preferred_element_type=jnp.float32)
        + b2_ref[...], 0.0)
    efc = jnp.dot(ef, w3c_ref[...], preferred_element_type=jnp.float32)
    hid = jnp.maximum(g_ref[...] + efc, 0.0)
    o_ref[...] = jnp.dot(hid, w4_ref[...],
                         preferred_element_type=jnp.float32) + b4_ref[...]


def kernel(x, edge_index, edge_attr, W1, b1, W2, b2, Wg, bg, W3, b3, W4, b4):
    src2 = edge_index[0].reshape(NW, NCH, C).astype(jnp.int32)
    dst2 = edge_index[1].reshape(NW, NCH, C).astype(jnp.int32)

    degp = _deg_kernel(dst2)
    degc = degp.T

    u = pl.pallas_call(
        _node1_body,
        out_shape=jax.ShapeDtypeStruct((NP, H), jnp.float32),
    )(x, W1, b1.reshape(1, H), Wg, degc)

    sp = _scatter_kernel(src2, dst2, u)

    a_nodes, b_nodes = pl.pallas_call(
        _node2_body,
        out_shape=[
            jax.ShapeDtypeStruct((NP, H), jnp.float32),
            jax.ShapeDtypeStruct((NP, H), jnp.float32),
        ],
    )(sp, u, degc, bg.reshape(1, H), W3[:H], W3[H:2 * H], b3.reshape(1, H))

    g = _gather_kernel(src2, dst2, a_nodes, b_nodes)

    out = pl.pallas_call(
        _edge_body,
        grid=(E // BLK,),
        in_specs=[
            pl.BlockSpec((BLK, H), lambda i: (i, 0)),
            pl.BlockSpec((BLK, DE), lambda i: (i, 0)),
            pl.BlockSpec((DE, H), lambda i: (0, 0)),
            pl.BlockSpec((1, H), lambda i: (0, 0)),
            pl.BlockSpec((H, H), lambda i: (0, 0)),
            pl.BlockSpec((H, 1), lambda i: (0, 0)),
            pl.BlockSpec((1, 1), lambda i: (0, 0)),
        ],
        out_specs=pl.BlockSpec((BLK, 1), lambda i: (i, 0)),
        out_shape=jax.ShapeDtypeStruct((E, 1), jnp.float32),
    )(g, edge_attr, W2, b2.reshape(1, H), W3[2 * H:], W4, b4.reshape(1, 1))

    return out.reshape(-1)


# speedup vs baseline: 12.0742x; 1.0727x over previous
import functools

import jax
import jax.numpy as jnp
from jax import lax
from jax.experimental import pallas as pl
from jax.experimental.pallas import tpu as pltpu
from jax.experimental.pallas import tpu_sc as plsc

N, E, D, DE, H = 10000, 320000, 128, 16, 64
NP = 10240
NC, NS = 2, 16
NW = NC * NS
C = 80
EW = E // NW
NCH = EW // C
RWP = NP // NS

_mesh = plsc.VectorSubcoreMesh(core_axis_name="c", subcore_axis_name="s")


@functools.partial(
    pl.kernel,
    out_type=jax.ShapeDtypeStruct((NC, NP), jnp.float32),
    mesh=_mesh,
    scratch_types=[
        pltpu.VMEM((NCH, C), jnp.int32),
        pltpu.VMEM((C,), jnp.float32),
        pltpu.VMEM((RWP,), jnp.float32),
        pltpu.VMEM_SHARED((NP,), jnp.float32),
    ],
    compiler_params=pltpu.CompilerParams(use_tc_tiling_on_sc=False),
)
def _deg_kernel(dst2, deg_hbm, idx_v, ones_v, zbuf_v, deg_s):
    cid = lax.axis_index("c")
    sid = lax.axis_index("s")
    wid = sid * NC + cid

    def z16(i, _):
        zbuf_v[pl.ds(i * 16, 16)] = jnp.zeros((16,), jnp.float32)
        return 0

    lax.fori_loop(0, RWP // 16, z16, 0)
    for k in range(C // 16):
        ones_v[pl.ds(k * 16, 16)] = jnp.ones((16,), jnp.float32)
    pltpu.sync_copy(zbuf_v, deg_s.at[pl.ds(sid * RWP, RWP)])
    pltpu.sync_copy(dst2.at[wid], idx_v)
    plsc.subcore_barrier()

    def body(j, _):
        pltpu.sync_copy(ones_v, deg_s.at[idx_v.at[j]], add=True)
        return 0

    lax.fori_loop(0, NCH, body, 0)
    plsc.subcore_barrier()
    pltpu.sync_copy(deg_s.at[pl.ds(sid * RWP, RWP)], zbuf_v)
    pltpu.sync_copy(zbuf_v, deg_hbm.at[cid, pl.ds(sid * RWP, RWP)])


@functools.partial(
    pl.kernel,
    out_type=jax.ShapeDtypeStruct((NC, NP, H), jnp.float32),
    mesh=_mesh,
    scratch_types=[
        pltpu.VMEM((NCH, C), jnp.int32),
        pltpu.VMEM((NCH, C), jnp.int32),
        pltpu.VMEM((C, H), jnp.float32),
        pltpu.VMEM((C, H), jnp.float32),
        pltpu.VMEM_SHARED((NP, H), jnp.float32),
        pltpu.SemaphoreType.DMA,
    ],
    compiler_params=pltpu.CompilerParams(use_tc_tiling_on_sc=False),
)
def _scatter_kernel(src2, dst2, u_hbm, s_hbm, sidx_v, didx_v, rows_v, nbuf_v,
                    s_s, sem):
    cid = lax.axis_index("c")
    sid = lax.axis_index("s")
    wid = sid * NC + cid

    def zrow(i, _):
        for k in range(H // 16):
            nbuf_v[i, pl.ds(k * 16, 16)] = jnp.zeros((16,), jnp.float32)
        return 0

    lax.fori_loop(0, C, zrow, 0)
    for t in range(RWP // C):
        pltpu.sync_copy(nbuf_v, s_s.at[pl.ds(sid * RWP + t * C, C)])
    pltpu.sync_copy(src2.at[wid], sidx_v)
    pltpu.sync_copy(dst2.at[wid], didx_v)
    plsc.subcore_barrier()

    def body(j, _):
        pltpu.async_copy(u_hbm.at[sidx_v.at[j]], rows_v, sem).wait()
        pltpu.sync_copy(rows_v, s_s.at[didx_v.at[j]], add=True)
        return 0

    lax.fori_loop(0, NCH, body, 0)
    plsc.subcore_barrier()
    for t in range(RWP // C):
        pltpu.sync_copy(s_s.at[pl.ds(sid * RWP + t * C, C)], nbuf_v)
        pltpu.sync_copy(nbuf_v, s_hbm.at[cid, pl.ds(sid * RWP + t * C, C)])


E2 = E // 2
R = 40
NCH3 = E2 // NW // R


@functools.partial(
    pl.kernel,
    out_type=jax.ShapeDtypeStruct((E2, 2 * H), jnp.float32),
    mesh=_mesh,
    scratch_types=[
        pltpu.VMEM((NCH3, R), jnp.int32),
        pltpu.VMEM((NCH3, R), jnp.int32),
        pltpu.VMEM((NCH3, R), jnp.int32),
        pltpu.VMEM((NCH3, R), jnp.int32),
        pltpu.VMEM((R, H), jnp.float32),
        pltpu.VMEM((R, H), jnp.float32),
        pltpu.VMEM((R, 2 * H), jnp.float32),
        pltpu.VMEM_SHARED((NP, H), jnp.float32),
        pltpu.VMEM_SHARED((NP, H), jnp.float32),
        pltpu.SemaphoreType.DMA,
        pltpu.SemaphoreType.DMA,
    ],
    compiler_params=pltpu.CompilerParams(use_tc_tiling_on_sc=False),
)
def _gather_kernel(sl2, dl2, sr2, dr2, a_hbm, b_hbm, g_hbm, sl_v, dl_v, sr_v,
                   dr_v, bufa_v, bufb_v, bufg_v, a_s, b_s, sema, semb):
    cid = lax.axis_index("c")
    sid = lax.axis_index("s")
    wid = sid * NC + cid
    pltpu.sync_copy(sl2.at[wid], sl_v)
    pltpu.sync_copy(dl2.at[wid], dl_v)
    pltpu.sync_copy(sr2.at[wid], sr_v)
    pltpu.sync_copy(dr2.at[wid], dr_v)
    for t in range(RWP // R):
        sl = pl.ds(sid * RWP + t * R, R)
        pltpu.sync_copy(a_hbm.at[sl], bufa_v)
        pltpu.sync_copy(bufa_v, a_s.at[sl])
        pltpu.sync_copy(b_hbm.at[sl], bufb_v)
        pltpu.sync_copy(bufb_v, b_s.at[sl])
    plsc.subcore_barrier()

    def body(j, _):
        ca = pltpu.async_copy(a_s.at[sl_v.at[j]], bufa_v, sema)
        cb = pltpu.async_copy(b_s.at[dl_v.at[j]], bufb_v, semb)
        ca.wait()
        cb.wait()

        def addl(i, _):
            for k in range(H // 16):
                sl = pl.ds(k * 16, 16)
                bufg_v[i, sl] = bufa_v[i, sl] + bufb_v[i, sl]
            return 0

        lax.fori_loop(0, R, addl, 0)
        ca2 = pltpu.async_copy(a_s.at[sr_v.at[j]], bufa_v, sema)
        cb2 = pltpu.async_copy(b_s.at[dr_v.at[j]], bufb_v, semb)
        ca2.wait()
        cb2.wait()

        def addr(i, _):
            for k in range(H // 16):
                sl = pl.ds(k * 16, 16)
                bufg_v[i, pl.ds(H + k * 16, 16)] = bufa_v[i, sl] + bufb_v[i, sl]
            return 0

        lax.fori_loop(0, R, addr, 0)
        pltpu.sync_copy(bufg_v, g_hbm.at[pl.ds((wid * NCH3 + j) * R, R)])
        return 0

    lax.fori_loop(0, NCH3, body, 0)


def _node1_body(x_ref, w1_ref, b1_ref, wg_ref, degc_ref, u_ref):
    h = jnp.maximum(
        jnp.dot(x_ref[...], w1_ref[...], preferred_element_type=jnp.float32)
        + b1_ref[...], 0.0)
    hw = jnp.dot(h, wg_ref[...], preferred_element_type=jnp.float32)
    d = degc_ref[...]
    dinv = lax.rsqrt(d[:, 0:1] + d[:, 1:2] + 1.0)
    u_ref[:N] = hw * dinv[:N, :]
    u_ref[N:] = jnp.zeros((NP - N, H), jnp.float32)


def _node2_body(sp_ref, u_ref, degc_ref, bg_ref, w3a_ref, w3b_ref, b3_ref,
                a_ref, b_ref):
    s = sp_ref[0] + sp_ref[1] + u_ref[...]
    d = degc_ref[...]
    dinv = lax.rsqrt(d[:, 0:1] + d[:, 1:2] + 1.0)
    agg = s * dinv + bg_ref[...]
    a_ref[...] = jnp.dot(agg, w3a_ref[...], preferred_element_type=jnp.float32)
    b_ref[...] = jnp.dot(agg, w3b_ref[...],
                         preferred_element_type=jnp.float32) + b3_ref[...]


BLK2 = 3200


def _edge_body(g_ref, eatl_ref, eatr_ref, w2_ref, b2_ref, w3c_ref, w4p_ref,
               b4_ref, o_ref):
    ea = jnp.concatenate([eatl_ref[...].T, eatr_ref[...].T], axis=1)
    ef2 = jnp.maximum(
        jnp.dot(ea, w2_ref[...], preferred_element_type=jnp.float32)
        + b2_ref[...], 0.0)
    efc2 = jnp.dot(ef2, w3c_ref[...], preferred_element_type=jnp.float32)
    hid2 = jnp.maximum(g_ref[...] + efc2, 0.0)
    o = jnp.dot(hid2, w4p_ref[...],
                preferred_element_type=jnp.float32) + b4_ref[...]
    o_ref[...] = o.T


def kernel(x, edge_index, edge_attr, W1, b1, W2, b2, Wg, bg, W3, b3, W4, b4):
    src_all = edge_index[0].astype(jnp.int32)
    dst_all = edge_index[1].astype(jnp.int32)
    src2 = src_all.reshape(NW, NCH, C)
    dst2 = dst_all.reshape(NW, NCH, C)
    sl2 = src_all[:E2].reshape(NW, NCH3, R)
    dl2 = dst_all[:E2].reshape(NW, NCH3, R)
    sr2 = src_all[E2:].reshape(NW, NCH3, R)
    dr2 = dst_all[E2:].reshape(NW, NCH3, R)

    degp = _deg_kernel(dst2)
    degc = degp.T

    u = pl.pallas_call(
        _node1_body,
        out_shape=jax.ShapeDtypeStruct((NP, H), jnp.float32),
    )(x, W1, b1.reshape(1, H), Wg, degc)

    sp = _scatter_kernel(src2, dst2, u)

    a_nodes, b_nodes = pl.pallas_call(
        _node2_body,
        out_shape=[
            jax.ShapeDtypeStruct((NP, H), jnp.float32),
            jax.ShapeDtypeStruct((NP, H), jnp.float32),
        ],
    )(sp, u, degc, bg.reshape(1, H), W3[:H], W3[H:2 * H], b3.reshape(1, H))

    g2 = _gather_kernel(sl2, dl2, sr2, dr2, a_nodes, b_nodes)
    eat = edge_attr.T
    z = jnp.zeros((H, 1), jnp.float32)
    w4p = jnp.concatenate([jnp.concatenate([W4, z], axis=1),
                           jnp.concatenate([z, W4], axis=1)], axis=0)
    zde = jnp.zeros((DE, H), jnp.float32)
    w2p = jnp.concatenate([jnp.concatenate([W2, zde], axis=1),
                           jnp.concatenate([zde, W2], axis=1)], axis=0)
    b2p = jnp.concatenate([b2, b2]).reshape(1, 2 * H)
    zh = jnp.zeros((H, H), jnp.float32)
    w3c = W3[2 * H:]
    w3cp = jnp.concatenate([jnp.concatenate([w3c, zh], axis=1),
                            jnp.concatenate([zh, w3c], axis=1)], axis=0)

    out = pl.pallas_call(
        _edge_body,
        grid=(E2 // BLK2,),
        in_specs=[
            pl.BlockSpec((BLK2, 2 * H), lambda i: (i, 0)),
            pl.BlockSpec((DE, BLK2), lambda i: (0, i)),
            pl.BlockSpec((DE, BLK2), lambda i: (0, i + E2 // BLK2)),
            pl.BlockSpec((2 * DE, 2 * H), lambda i: (0, 0)),
            pl.BlockSpec((1, 2 * H), lambda i: (0, 0)),
            pl.BlockSpec((2 * H, 2 * H), lambda i: (0, 0)),
            pl.BlockSpec((2 * H, 2), lambda i: (0, 0)),
            pl.BlockSpec((1, 1), lambda i: (0, 0)),
        ],
        out_specs=pl.BlockSpec((2, BLK2), lambda i: (0, i)),
        out_shape=jax.ShapeDtypeStruct((2, E2), jnp.float32),
    )(g2, eat, eat, w2p, b2p, w3cp, w4p, b4.reshape(1, 1))

    return out.reshape(-1)


# speedup vs baseline: 13.6040x; 1.1267x over previous
import functools

import jax
import jax.numpy as jnp
from jax import lax
from jax.experimental import pallas as pl
from jax.experimental.pallas import tpu as pltpu
from jax.experimental.pallas import tpu_sc as plsc

N, E, D, DE, H = 10000, 320000, 128, 16, 64
NP = 10240
NC, NS = 2, 16
NW = NC * NS
C = 80
EW = E // NW
NCH = EW // C
RWP = NP // NS

_mesh = plsc.VectorSubcoreMesh(core_axis_name="c", subcore_axis_name="s")


@functools.partial(
    pl.kernel,
    out_type=jax.ShapeDtypeStruct((NC, NP), jnp.float32),
    mesh=_mesh,
    scratch_types=[
        pltpu.VMEM((NCH, C), jnp.int32),
        pltpu.VMEM((C,), jnp.float32),
        pltpu.VMEM((RWP,), jnp.float32),
        pltpu.VMEM_SHARED((NP,), jnp.float32),
    ],
    compiler_params=pltpu.CompilerParams(use_tc_tiling_on_sc=False),
)
def _deg_kernel(dst2, deg_hbm, idx_v, ones_v, zbuf_v, deg_s):
    cid = lax.axis_index("c")
    sid = lax.axis_index("s")
    wid = sid * NC + cid

    def z16(i, _):
        zbuf_v[pl.ds(i * 16, 16)] = jnp.zeros((16,), jnp.float32)
        return 0

    lax.fori_loop(0, RWP // 16, z16, 0)
    for k in range(C // 16):
        ones_v[pl.ds(k * 16, 16)] = jnp.ones((16,), jnp.float32)
    pltpu.sync_copy(zbuf_v, deg_s.at[pl.ds(sid * RWP, RWP)])
    pltpu.sync_copy(dst2.at[wid], idx_v)
    plsc.subcore_barrier()

    def body(j, _):
        pltpu.sync_copy(ones_v, deg_s.at[idx_v.at[j]], add=True)
        return 0

    lax.fori_loop(0, NCH, body, 0)
    plsc.subcore_barrier()
    pltpu.sync_copy(deg_s.at[pl.ds(sid * RWP, RWP)], zbuf_v)
    pltpu.sync_copy(zbuf_v, deg_hbm.at[cid, pl.ds(sid * RWP, RWP)])


C2 = 100
NCH2 = EW // C2
SB = 80


@functools.partial(
    pl.kernel,
    out_type=jax.ShapeDtypeStruct((NC, NP, H), jnp.float32),
    mesh=_mesh,
    scratch_types=[
        pltpu.VMEM((NCH2, C2), jnp.int32),
        pltpu.VMEM((NCH2, C2), jnp.int32),
        pltpu.VMEM((C2, H), jnp.float32),
        pltpu.VMEM((C2, H), jnp.float32),
        pltpu.VMEM((SB, H), jnp.float32),
        pltpu.VMEM_SHARED((NP, H), jnp.float32),
        pltpu.VMEM_SHARED((NP, H), jnp.float32),
        pltpu.SemaphoreType.DMA,
        pltpu.SemaphoreType.DMA,
    ],
    compiler_params=pltpu.CompilerParams(use_tc_tiling_on_sc=False),
)
def _scatter_kernel(src2, dst2, u_hbm, s_hbm, sidx_v, didx_v, rows0_v, rows1_v,
                    nbuf_v, u_s, s_s, sem0, sem1):
    cid = lax.axis_index("c")
    sid = lax.axis_index("s")
    wid = sid * NC + cid

    def zrow(i, _):
        for k in range(H // 16):
            nbuf_v[i, pl.ds(k * 16, 16)] = jnp.zeros((16,), jnp.float32)
        return 0

    lax.fori_loop(0, SB, zrow, 0)
    for t in range(RWP // SB):
        sl = pl.ds(sid * RWP + t * SB, SB)
        pltpu.sync_copy(nbuf_v, s_s.at[sl])
        pltpu.sync_copy(u_hbm.at[sl], nbuf_v)
        pltpu.sync_copy(nbuf_v, u_s.at[sl])
    pltpu.sync_copy(src2.at[wid], sidx_v)
    pltpu.sync_copy(dst2.at[wid], didx_v)
    plsc.subcore_barrier()

    pltpu.async_copy(u_s.at[sidx_v.at[0]], rows0_v, sem0)
    pltpu.async_copy(u_s.at[sidx_v.at[1]], rows1_v, sem1)

    def body(i, _):
        j = 2 * i
        pltpu.make_async_copy(u_hbm.at[pl.ds(0, C2)], rows0_v, sem0).wait()
        pltpu.sync_copy(rows0_v, s_s.at[didx_v.at[j]], add=True)
        pltpu.async_copy(u_s.at[sidx_v.at[(j + 2) % NCH2]], rows0_v, sem0)
        pltpu.make_async_copy(u_hbm.at[pl.ds(0, C2)], rows1_v, sem1).wait()
        pltpu.sync_copy(rows1_v, s_s.at[didx_v.at[j + 1]], add=True)
        pltpu.async_copy(u_s.at[sidx_v.at[(j + 3) % NCH2]], rows1_v, sem1)
        return 0

    lax.fori_loop(0, NCH2 // 2, body, 0)
    pltpu.make_async_copy(u_hbm.at[pl.ds(0, C2)], rows0_v, sem0).wait()
    pltpu.make_async_copy(u_hbm.at[pl.ds(0, C2)], rows1_v, sem1).wait()
    plsc.subcore_barrier()
    for t in range(RWP // SB):
        sl = pl.ds(sid * RWP + t * SB, SB)
        pltpu.sync_copy(s_s.at[sl], nbuf_v)
        pltpu.sync_copy(nbuf_v, s_hbm.at[cid, sl])


E2 = E // 2
R = 64
E2P = 163840
NCH3 = E2P // NW // R


@functools.partial(
    pl.kernel,
    out_type=jax.ShapeDtypeStruct((E2P, 2 * H), jnp.float32),
    mesh=_mesh,
    scratch_types=[
        pltpu.VMEM((NCH3, R), jnp.int32),
        pltpu.VMEM((NCH3, R), jnp.int32),
        pltpu.VMEM((NCH3, R), jnp.int32),
        pltpu.VMEM((NCH3, R), jnp.int32),
        pltpu.VMEM((R, H), jnp.float32),
        pltpu.VMEM((R, H), jnp.float32),
        pltpu.VMEM((R, H), jnp.float32),
        pltpu.VMEM((R, H), jnp.float32),
        pltpu.VMEM((R, 2 * H), jnp.float32),
        pltpu.VMEM_SHARED((NP, H), jnp.float32),
        pltpu.VMEM_SHARED((NP, H), jnp.float32),
        pltpu.SemaphoreType.DMA,
        pltpu.SemaphoreType.DMA,
        pltpu.SemaphoreType.DMA,
        pltpu.SemaphoreType.DMA,
    ],
    compiler_params=pltpu.CompilerParams(use_tc_tiling_on_sc=False),
)
def _gather_kernel(sl2, dl2, sr2, dr2, a_hbm, b_hbm, g_hbm, sl_v, dl_v, sr_v,
                   dr_v, bufa_v, bufb_v, bufc_v, bufd_v, bufg_v, a_s, b_s,
                   sema, semb, semc, semd):
    cid = lax.axis_index("c")
    sid = lax.axis_index("s")
    wid = sid * NC + cid
    pltpu.sync_copy(sl2.at[wid], sl_v)
    pltpu.sync_copy(dl2.at[wid], dl_v)
    pltpu.sync_copy(sr2.at[wid], sr_v)
    pltpu.sync_copy(dr2.at[wid], dr_v)
    for t in range(RWP // R):
        sl = pl.ds(sid * RWP + t * R, R)
        pltpu.sync_copy(a_hbm.at[sl], bufa_v)
        pltpu.sync_copy(bufa_v, a_s.at[sl])
        pltpu.sync_copy(b_hbm.at[sl], bufb_v)
        pltpu.sync_copy(bufb_v, b_s.at[sl])
    plsc.subcore_barrier()

    def body(j, _):
        ca = pltpu.async_copy(a_s.at[sl_v.at[j]], bufa_v, sema)
        cb = pltpu.async_copy(b_s.at[dl_v.at[j]], bufb_v, semb)
        cc = pltpu.async_copy(a_s.at[sr_v.at[j]], bufc_v, semc)
        cd = pltpu.async_copy(b_s.at[dr_v.at[j]], bufd_v, semd)
        ca.wait()
        cb.wait()
        cc.wait()
        cd.wait()

        def addrow(i, _):
            for k in range(H // 16):
                sl = pl.ds(k * 16, 16)
                bufg_v[i, sl] = bufa_v[i, sl] + bufb_v[i, sl]
                bufg_v[i, pl.ds(H + k * 16, 16)] = (
                    bufc_v[i, sl] + bufd_v[i, sl])
            return 0

        lax.fori_loop(0, R, addrow, 0)
        pltpu.sync_copy(bufg_v, g_hbm.at[pl.ds((wid * NCH3 + j) * R, R)])
        return 0

    lax.fori_loop(0, NCH3, body, 0)


def _node1_body(x_ref, w1_ref, b1_ref, wg_ref, degc_ref, u_ref):
    h = jnp.maximum(
        jnp.dot(x_ref[...], w1_ref[...], preferred_element_type=jnp.float32)
        + b1_ref[...], 0.0)
    hw = jnp.dot(h, wg_ref[...], preferred_element_type=jnp.float32)
    d = degc_ref[...]
    dinv = lax.rsqrt(d[:, 0:1] + d[:, 1:2] + 1.0)
    u_ref[:N] = hw * dinv[:N, :]
    u_ref[N:] = jnp.zeros((NP - N, H), jnp.float32)


def _node2_body(sp_ref, u_ref, degc_ref, bg_ref, w3a_ref, w3b_ref, b3_ref,
                a_ref, b_ref):
    s = sp_ref[0] + sp_ref[1] + u_ref[...]
    d = degc_ref[...]
    dinv = lax.rsqrt(d[:, 0:1] + d[:, 1:2] + 1.0)
    agg = s * dinv + bg_ref[...]
    a_ref[...] = jnp.dot(agg, w3a_ref[...], preferred_element_type=jnp.float32)
    b_ref[...] = jnp.dot(agg, w3b_ref[...],
                         preferred_element_type=jnp.float32) + b3_ref[...]


BLK2 = 3200


def _edge_body(g_ref, eatl_ref, eatr_ref, w2_ref, b2_ref, w3c_ref, w4p_ref,
               b4_ref, o_ref):
    ea = jnp.concatenate([eatl_ref[...].T, eatr_ref[...].T], axis=1)
    ef2 = jnp.maximum(
        jnp.dot(ea, w2_ref[...], preferred_element_type=jnp.float32)
        + b2_ref[...], 0.0)
    efc2 = jnp.dot(ef2, w3c_ref[...], preferred_element_type=jnp.float32)
    hid2 = jnp.maximum(g_ref[...] + efc2, 0.0)
    o = jnp.dot(hid2, w4p_ref[...],
                preferred_element_type=jnp.float32) + b4_ref[...]
    o_ref[...] = o.T


def kernel(x, edge_index, edge_attr, W1, b1, W2, b2, Wg, bg, W3, b3, W4, b4):
    src_all = edge_index[0].astype(jnp.int32)
    dst_all = edge_index[1].astype(jnp.int32)
    src2 = src_all.reshape(NW, NCH, C)
    dst2 = dst_all.reshape(NW, NCH, C)
    srcc = src_all.reshape(NW, NCH2, C2)
    dstc = dst_all.reshape(NW, NCH2, C2)
    pad = (jnp.arange(E2P - E2, dtype=jnp.int32) * 131) % N
    sl2 = jnp.concatenate([src_all[:E2], pad]).reshape(NW, NCH3, R)
    dl2 = jnp.concatenate([dst_all[:E2], pad]).reshape(NW, NCH3, R)
    sr2 = jnp.concatenate([src_all[E2:], pad]).reshape(NW, NCH3, R)
    dr2 = jnp.concatenate([dst_all[E2:], pad]).reshape(NW, NCH3, R)

    degp = _deg_kernel(dst2)
    degc = degp.T

    u = pl.pallas_call(
        _node1_body,
        out_shape=jax.ShapeDtypeStruct((NP, H), jnp.float32),
    )(x, W1, b1.reshape(1, H), Wg, degc)

    sp = _scatter_kernel(srcc, dstc, u)

    a_nodes, b_nodes = pl.pallas_call(
        _node2_body,
        out_shape=[
            jax.ShapeDtypeStruct((NP, H), jnp.float32),
            jax.ShapeDtypeStruct((NP, H), jnp.float32),
        ],
    )(sp, u, degc, bg.reshape(1, H), W3[:H], W3[H:2 * H], b3.reshape(1, H))

    g2 = _gather_kernel(sl2, dl2, sr2, dr2, a_nodes, b_nodes)
    eat = edge_attr.T
    z = jnp.zeros((H, 1), jnp.float32)
    w4p = jnp.concatenate([jnp.concatenate([W4, z], axis=1),
                           jnp.concatenate([z, W4], axis=1)], axis=0)
    zde = jnp.zeros((DE, H), jnp.float32)
    w2p = jnp.concatenate([jnp.concatenate([W2, zde], axis=1),
                           jnp.concatenate([zde, W2], axis=1)], axis=0)
    b2p = jnp.concatenate([b2, b2]).reshape(1, 2 * H)
    zh = jnp.zeros((H, H), jnp.float32)
    w3c = W3[2 * H:]
    w3cp = jnp.concatenate([jnp.concatenate([w3c, zh], axis=1),
                            jnp.concatenate([zh, w3c], axis=1)], axis=0)

    out = pl.pallas_call(
        _edge_body,
        grid=(E2 // BLK2,),
        in_specs=[
            pl.BlockSpec((BLK2, 2 * H), lambda i: (i, 0)),
            pl.BlockSpec((DE, BLK2), lambda i: (0, i)),
            pl.BlockSpec((DE, BLK2), lambda i: (0, i + E2 // BLK2)),
            pl.BlockSpec((2 * DE, 2 * H), lambda i: (0, 0)),
            pl.BlockSpec((1, 2 * H), lambda i: (0, 0)),
            pl.BlockSpec((2 * H, 2 * H), lambda i: (0, 0)),
            pl.BlockSpec((2 * H, 2), lambda i: (0, 0)),
            pl.BlockSpec((1, 1), lambda i: (0, 0)),
        ],
        out_specs=pl.BlockSpec((2, BLK2), lambda i: (0, i)),
        out_shape=jax.ShapeDtypeStruct((2, E2), jnp.float32),
    )(g2, eat, eat, w2p, b2p, w3cp, w4p, b4.reshape(1, 1))

    return out.reshape(-1)


# speedup vs baseline: 13.6221x; 1.0013x over previous
import functools

import jax
import jax.numpy as jnp
from jax import lax
from jax.experimental import pallas as pl
from jax.experimental.pallas import tpu as pltpu
from jax.experimental.pallas import tpu_sc as plsc

N, E, D, DE, H = 10000, 320000, 128, 16, 64
NP = 10240
NC, NS = 2, 16
NW = NC * NS
C = 80
EW = E // NW
NCH = EW // C
RWP = NP // NS

_mesh = plsc.VectorSubcoreMesh(core_axis_name="c", subcore_axis_name="s")


@functools.partial(
    pl.kernel,
    out_type=jax.ShapeDtypeStruct((NC, NP), jnp.float32),
    mesh=_mesh,
    scratch_types=[
        pltpu.VMEM((NCH, C), jnp.int32),
        pltpu.VMEM((C,), jnp.float32),
        pltpu.VMEM((RWP,), jnp.float32),
        pltpu.VMEM_SHARED((NP,), jnp.float32),
    ],
    compiler_params=pltpu.CompilerParams(use_tc_tiling_on_sc=False),
)
def _deg_kernel(dst2, deg_hbm, idx_v, ones_v, zbuf_v, deg_s):
    cid = lax.axis_index("c")
    sid = lax.axis_index("s")
    wid = sid * NC + cid

    def z16(i, _):
        zbuf_v[pl.ds(i * 16, 16)] = jnp.zeros((16,), jnp.float32)
        return 0

    lax.fori_loop(0, RWP // 16, z16, 0)
    for k in range(C // 16):
        ones_v[pl.ds(k * 16, 16)] = jnp.ones((16,), jnp.float32)
    pltpu.sync_copy(zbuf_v, deg_s.at[pl.ds(sid * RWP, RWP)])
    pltpu.sync_copy(dst2.at[wid], idx_v)
    plsc.subcore_barrier()

    def body(j, _):
        pltpu.sync_copy(ones_v, deg_s.at[idx_v.at[j]], add=True)
        return 0

    lax.fori_loop(0, NCH, body, 0)
    plsc.subcore_barrier()
    pltpu.sync_copy(deg_s.at[pl.ds(sid * RWP, RWP)], zbuf_v)
    pltpu.sync_copy(zbuf_v, deg_hbm.at[cid, pl.ds(sid * RWP, RWP)])


C2 = 100
NCH2 = EW // C2
SB = 80


@functools.partial(
    pl.kernel,
    out_type=jax.ShapeDtypeStruct((NC, NP, H), jnp.float32),
    mesh=_mesh,
    scratch_types=[
        pltpu.VMEM((NCH2, C2), jnp.int32),
        pltpu.VMEM((NCH2, C2), jnp.int32),
        pltpu.VMEM((C2, H), jnp.float32),
        pltpu.VMEM((C2, H), jnp.float32),
        pltpu.VMEM((SB, H), jnp.float32),
        pltpu.VMEM_SHARED((NP, H), jnp.float32),
        pltpu.VMEM_SHARED((NP, H), jnp.float32),
        pltpu.SemaphoreType.DMA,
        pltpu.SemaphoreType.DMA,
    ],
    compiler_params=pltpu.CompilerParams(use_tc_tiling_on_sc=False),
)
def _scatter_kernel(src2, dst2, u_hbm, s_hbm, sidx_v, didx_v, rows0_v, rows1_v,
                    nbuf_v, u_s, s_s, sem0, sem1):
    cid = lax.axis_index("c")
    sid = lax.axis_index("s")
    wid = sid * NC + cid

    def zrow(i, _):
        for k in range(H // 16):
            nbuf_v[i, pl.ds(k * 16, 16)] = jnp.zeros((16,), jnp.float32)
        return 0

    lax.fori_loop(0, SB, zrow, 0)
    for t in range(RWP // SB):
        pltpu.sync_copy(nbuf_v, s_s.at[pl.ds(sid * RWP + t * SB, SB)])
    for t in range(RWP // SB):
        sl = pl.ds(sid * RWP + t * SB, SB)
        pltpu.sync_copy(u_hbm.at[sl], nbuf_v)
        pltpu.sync_copy(nbuf_v, u_s.at[sl])
    pltpu.sync_copy(src2.at[wid], sidx_v)
    pltpu.sync_copy(dst2.at[wid], didx_v)
    plsc.subcore_barrier()

    pltpu.async_copy(u_s.at[sidx_v.at[0]], rows0_v, sem0)
    pltpu.async_copy(u_s.at[sidx_v.at[1]], rows1_v, sem1)

    def body(i, _):
        j = 2 * i
        pltpu.make_async_copy(u_hbm.at[pl.ds(0, C2)], rows0_v, sem0).wait()
        pltpu.sync_copy(rows0_v, s_s.at[didx_v.at[j]], add=True)
        pltpu.async_copy(u_s.at[sidx_v.at[(j + 2) % NCH2]], rows0_v, sem0)
        pltpu.make_async_copy(u_hbm.at[pl.ds(0, C2)], rows1_v, sem1).wait()
        pltpu.sync_copy(rows1_v, s_s.at[didx_v.at[j + 1]], add=True)
        pltpu.async_copy(u_s.at[sidx_v.at[(j + 3) % NCH2]], rows1_v, sem1)
        return 0

    lax.fori_loop(0, NCH2 // 2, body, 0)
    pltpu.make_async_copy(u_hbm.at[pl.ds(0, C2)], rows0_v, sem0).wait()
    pltpu.make_async_copy(u_hbm.at[pl.ds(0, C2)], rows1_v, sem1).wait()
    plsc.subcore_barrier()
    for t in range(RWP // SB):
        sl = pl.ds(sid * RWP + t * SB, SB)
        pltpu.sync_copy(s_s.at[sl], nbuf_v)
        pltpu.sync_copy(nbuf_v, s_hbm.at[cid, sl])


E2 = E // 2
R = 64
E2P = 163840
NCH3 = E2P // NW // R


@functools.partial(
    pl.kernel,
    out_type=jax.ShapeDtypeStruct((E2P, 2 * H), jnp.float32),
    mesh=_mesh,
    scratch_types=[
        pltpu.VMEM((NCH3, R), jnp.int32),
        pltpu.VMEM((NCH3, R), jnp.int32),
        pltpu.VMEM((NCH3, R), jnp.int32),
        pltpu.VMEM((NCH3, R), jnp.int32),
        pltpu.VMEM((R, H), jnp.float32),
        pltpu.VMEM((R, H), jnp.float32),
        pltpu.VMEM((R, H), jnp.float32),
        pltpu.VMEM((R, H), jnp.float32),
        pltpu.VMEM((R, 2 * H), jnp.float32),
        pltpu.VMEM_SHARED((NP, H), jnp.float32),
        pltpu.VMEM_SHARED((NP, H), jnp.float32),
        pltpu.SemaphoreType.DMA,
        pltpu.SemaphoreType.DMA,
        pltpu.SemaphoreType.DMA,
        pltpu.SemaphoreType.DMA,
    ],
    compiler_params=pltpu.CompilerParams(use_tc_tiling_on_sc=False),
)
def _gather_kernel(sl2, dl2, sr2, dr2, a_hbm, b_hbm, g_hbm, sl_v, dl_v, sr_v,
                   dr_v, bufa_v, bufb_v, bufc_v, bufd_v, bufg_v, a_s, b_s,
                   sema, semb, semc, semd):
    cid = lax.axis_index("c")
    sid = lax.axis_index("s")
    wid = sid * NC + cid
    pltpu.sync_copy(sl2.at[wid], sl_v)
    pltpu.sync_copy(dl2.at[wid], dl_v)
    pltpu.sync_copy(sr2.at[wid], sr_v)
    pltpu.sync_copy(dr2.at[wid], dr_v)
    for t in range(RWP // R):
        sl = pl.ds(sid * RWP + t * R, R)
        pltpu.sync_copy(a_hbm.at[sl], bufa_v)
        pltpu.sync_copy(bufa_v, a_s.at[sl])
        pltpu.sync_copy(b_hbm.at[sl], bufb_v)
        pltpu.sync_copy(bufb_v, b_s.at[sl])
    plsc.subcore_barrier()

    def body(j, _):
        ca = pltpu.async_copy(a_s.at[sl_v.at[j]], bufa_v, sema)
        cb = pltpu.async_copy(b_s.at[dl_v.at[j]], bufb_v, semb)
        cc = pltpu.async_copy(a_s.at[sr_v.at[j]], bufc_v, semc)
        cd = pltpu.async_copy(b_s.at[dr_v.at[j]], bufd_v, semd)
        ca.wait()
        cb.wait()
        cc.wait()
        cd.wait()

        def addrow(i, _):
            for k in range(H // 16):
                sl = pl.ds(k * 16, 16)
                bufg_v[i, sl] = bufa_v[i, sl] + bufb_v[i, sl]
                bufg_v[i, pl.ds(H + k * 16, 16)] = (
                    bufc_v[i, sl] + bufd_v[i, sl])
            return 0

        lax.fori_loop(0, R, addrow, 0)
        pltpu.sync_copy(bufg_v, g_hbm.at[pl.ds((wid * NCH3 + j) * R, R)])
        return 0

    lax.fori_loop(0, NCH3, body, 0)


def _node1_body(x_ref, w1_ref, b1_ref, wg_ref, degc_ref, u_ref):
    h = jnp.maximum(
        jnp.dot(x_ref[...], w1_ref[...], preferred_element_type=jnp.float32)
        + b1_ref[...], 0.0)
    hw = jnp.dot(h, wg_ref[...], preferred_element_type=jnp.float32)
    d = degc_ref[...]
    dinv = lax.rsqrt(d[:, 0:1] + d[:, 1:2] + 1.0)
    u_ref[:N] = hw * dinv[:N, :]
    u_ref[N:] = jnp.zeros((NP - N, H), jnp.float32)


def _node2_body(sp_ref, u_ref, degc_ref, bg_ref, w3a_ref, w3b_ref, b3_ref,
                a_ref, b_ref):
    s = sp_ref[0] + sp_ref[1] + u_ref[...]
    d = degc_ref[...]
    dinv = lax.rsqrt(d[:, 0:1] + d[:, 1:2] + 1.0)
    agg = s * dinv + bg_ref[...]
    a_ref[...] = jnp.dot(agg, w3a_ref[...], preferred_element_type=jnp.float32)
    b_ref[...] = jnp.dot(agg, w3b_ref[...],
                         preferred_element_type=jnp.float32) + b3_ref[...]


BLK2 = 3200


def _edge_body(g_ref, eatl_ref, eatr_ref, w2_ref, b2_ref, w3c_ref, w4p_ref,
               b4_ref, o_ref):
    ea = jnp.concatenate([eatl_ref[...].T, eatr_ref[...].T], axis=1)
    ef2 = jnp.maximum(
        jnp.dot(ea, w2_ref[...], preferred_element_type=jnp.float32)
        + b2_ref[...], 0.0)
    efc2 = jnp.dot(ef2, w3c_ref[...], preferred_element_type=jnp.float32)
    hid2 = jnp.maximum(g_ref[...] + efc2, 0.0)
    o = jnp.dot(hid2, w4p_ref[...],
                preferred_element_type=jnp.float32) + b4_ref[...]
    o_ref[...] = o.T


def kernel(x, edge_index, edge_attr, W1, b1, W2, b2, Wg, bg, W3, b3, W4, b4):
    src_all = edge_index[0].astype(jnp.int32)
    dst_all = edge_index[1].astype(jnp.int32)
    src2 = src_all.reshape(NW, NCH, C)
    dst2 = dst_all.reshape(NW, NCH, C)
    srcc = src_all.reshape(NW, NCH2, C2)
    dstc = dst_all.reshape(NW, NCH2, C2)
    pad = (jnp.arange(E2P - E2, dtype=jnp.int32) * 131) % N
    sl2 = jnp.concatenate([src_all[:E2], pad]).reshape(NW, NCH3, R)
    dl2 = jnp.concatenate([dst_all[:E2], pad]).reshape(NW, NCH3, R)
    sr2 = jnp.concatenate([src_all[E2:], pad]).reshape(NW, NCH3, R)
    dr2 = jnp.concatenate([dst_all[E2:], pad]).reshape(NW, NCH3, R)

    degp = _deg_kernel(dst2)
    degc = degp.T

    u = pl.pallas_call(
        _node1_body,
        out_shape=jax.ShapeDtypeStruct((NP, H), jnp.float32),
    )(x, W1, b1.reshape(1, H), Wg, degc)

    sp = _scatter_kernel(srcc, dstc, u)

    a_nodes, b_nodes = pl.pallas_call(
        _node2_body,
        out_shape=[
            jax.ShapeDtypeStruct((NP, H), jnp.float32),
            jax.ShapeDtypeStruct((NP, H), jnp.float32),
        ],
    )(sp, u, degc, bg.reshape(1, H), W3[:H], W3[H:2 * H], b3.reshape(1, H))

    g2 = _gather_kernel(sl2, dl2, sr2, dr2, a_nodes, b_nodes)
    eat = edge_attr.T
    z = jnp.zeros((H, 1), jnp.float32)
    w4p = jnp.concatenate([jnp.concatenate([W4, z], axis=1),
                           jnp.concatenate([z, W4], axis=1)], axis=0)
    zde = jnp.zeros((DE, H), jnp.float32)
    w2p = jnp.concatenate([jnp.concatenate([W2, zde], axis=1),
                           jnp.concatenate([zde, W2], axis=1)], axis=0)
    b2p = jnp.concatenate([b2, b2]).reshape(1, 2 * H)
    zh = jnp.zeros((H, H), jnp.float32)
    w3c = W3[2 * H:]
    w3cp = jnp.concatenate([jnp.concatenate([w3c, zh], axis=1),
                            jnp.concatenate([zh, w3c], axis=1)], axis=0)

    out = pl.pallas_call(
        _edge_body,
        grid=(E2 // BLK2,),
        in_specs=[
            pl.BlockSpec((BLK2, 2 * H), lambda i: (i, 0)),
            pl.BlockSpec((DE, BLK2), lambda i: (0, i)),
            pl.BlockSpec((DE, BLK2), lambda i: (0, i + E2 // BLK2)),
            pl.BlockSpec((2 * DE, 2 * H), lambda i: (0, 0)),
            pl.BlockSpec((1, 2 * H), lambda i: (0, 0)),
            pl.BlockSpec((2 * H, 2 * H), lambda i: (0, 0)),
            pl.BlockSpec((2 * H, 2), lambda i: (0, 0)),
            pl.BlockSpec((1, 1), lambda i: (0, 0)),
        ],
        out_specs=pl.BlockSpec((2, BLK2), lambda i: (0, i)),
        out_shape=jax.ShapeDtypeStruct((2, E2), jnp.float32),
    )(g2, eat, eat, w2p, b2p, w3cp, w4p, b4.reshape(1, 1))

    return out.reshape(-1)


# speedup vs baseline: 13.8505x; 1.0168x over previous
import functools

import jax
import jax.numpy as jnp
from jax import lax
from jax.experimental import pallas as pl
from jax.experimental.pallas import tpu as pltpu
from jax.experimental.pallas import tpu_sc as plsc

N, E, D, DE, H = 10000, 320000, 128, 16, 64
NP = 10240
NC, NS = 2, 16
NW = NC * NS
C = 80
EW = E // NW
NCH = EW // C
RWP = NP // NS

_mesh = plsc.VectorSubcoreMesh(core_axis_name="c", subcore_axis_name="s")


@functools.partial(
    pl.kernel,
    out_type=jax.ShapeDtypeStruct((NC, NP), jnp.float32),
    mesh=_mesh,
    scratch_types=[
        pltpu.VMEM((NCH, C), jnp.int32),
        pltpu.VMEM((C,), jnp.float32),
        pltpu.VMEM((RWP,), jnp.float32),
        pltpu.VMEM_SHARED((NP,), jnp.float32),
    ],
    compiler_params=pltpu.CompilerParams(use_tc_tiling_on_sc=False),
)
def _deg_kernel(dst2, deg_hbm, idx_v, ones_v, zbuf_v, deg_s):
    cid = lax.axis_index("c")
    sid = lax.axis_index("s")
    wid = sid * NC + cid

    def z16(i, _):
        zbuf_v[pl.ds(i * 16, 16)] = jnp.zeros((16,), jnp.float32)
        return 0

    lax.fori_loop(0, RWP // 16, z16, 0)
    for k in range(C // 16):
        ones_v[pl.ds(k * 16, 16)] = jnp.ones((16,), jnp.float32)
    pltpu.sync_copy(zbuf_v, deg_s.at[pl.ds(sid * RWP, RWP)])
    pltpu.sync_copy(dst2.at[wid], idx_v)
    plsc.subcore_barrier()

    def body(j, _):
        pltpu.sync_copy(ones_v, deg_s.at[idx_v.at[j]], add=True)
        return 0

    lax.fori_loop(0, NCH, body, 0)
    plsc.subcore_barrier()
    pltpu.sync_copy(deg_s.at[pl.ds(sid * RWP, RWP)], zbuf_v)
    pltpu.sync_copy(zbuf_v, deg_hbm.at[cid, pl.ds(sid * RWP, RWP)])


C2 = 100
NCH2 = EW // C2
SB = 80


@functools.partial(
    pl.kernel,
    out_type=jax.ShapeDtypeStruct((NC, NP, H), jnp.float32),
    mesh=_mesh,
    scratch_types=[
        pltpu.VMEM((NCH2, C2), jnp.int32),
        pltpu.VMEM((NCH2, C2), jnp.int32),
        pltpu.VMEM((C2, H), jnp.float32),
        pltpu.VMEM((C2, H), jnp.float32),
        pltpu.VMEM((SB, H), jnp.float32),
        pltpu.VMEM_SHARED((NP, H), jnp.float32),
        pltpu.VMEM_SHARED((NP, H), jnp.float32),
        pltpu.SemaphoreType.DMA,
        pltpu.SemaphoreType.DMA,
    ],
    compiler_params=pltpu.CompilerParams(use_tc_tiling_on_sc=False),
)
def _scatter_kernel(src2, dst2, u_hbm, s_hbm, sidx_v, didx_v, rows0_v, rows1_v,
                    nbuf_v, u_s, s_s, sem0, sem1):
    cid = lax.axis_index("c")
    sid = lax.axis_index("s")
    wid = sid * NC + cid

    def zrow(i, _):
        for k in range(H // 16):
            nbuf_v[i, pl.ds(k * 16, 16)] = jnp.zeros((16,), jnp.float32)
        return 0

    lax.fori_loop(0, SB, zrow, 0)
    for t in range(RWP // SB):
        pltpu.sync_copy(nbuf_v, s_s.at[pl.ds(sid * RWP + t * SB, SB)])
    for t in range(RWP // SB):
        sl = pl.ds(sid * RWP + t * SB, SB)
        pltpu.sync_copy(u_hbm.at[sl], nbuf_v)
        pltpu.sync_copy(nbuf_v, u_s.at[sl])
    pltpu.sync_copy(src2.at[wid], sidx_v)
    pltpu.sync_copy(dst2.at[wid], didx_v)
    plsc.subcore_barrier()

    pltpu.async_copy(u_s.at[sidx_v.at[0]], rows0_v, sem0)
    pltpu.async_copy(u_s.at[sidx_v.at[1]], rows1_v, sem1)

    def body(i, _):
        j = 2 * i
        pltpu.make_async_copy(u_hbm.at[pl.ds(0, C2)], rows0_v, sem0).wait()
        pltpu.sync_copy(rows0_v, s_s.at[didx_v.at[j]], add=True)
        pltpu.async_copy(u_s.at[sidx_v.at[(j + 2) % NCH2]], rows0_v, sem0)
        pltpu.make_async_copy(u_hbm.at[pl.ds(0, C2)], rows1_v, sem1).wait()
        pltpu.sync_copy(rows1_v, s_s.at[didx_v.at[j + 1]], add=True)
        pltpu.async_copy(u_s.at[sidx_v.at[(j + 3) % NCH2]], rows1_v, sem1)
        return 0

    lax.fori_loop(0, NCH2 // 2, body, 0)
    pltpu.make_async_copy(u_hbm.at[pl.ds(0, C2)], rows0_v, sem0).wait()
    pltpu.make_async_copy(u_hbm.at[pl.ds(0, C2)], rows1_v, sem1).wait()
    plsc.subcore_barrier()
    for t in range(RWP // SB):
        sl = pl.ds(sid * RWP + t * SB, SB)
        pltpu.sync_copy(s_s.at[sl], nbuf_v)
        pltpu.sync_copy(nbuf_v, s_hbm.at[cid, sl])


E2 = E // 2
R = 64
E2P = 163840
NCH3 = E2P // NW // R


@functools.partial(
    pl.kernel,
    out_type=jax.ShapeDtypeStruct((E2P, 2 * H), jnp.float32),
    mesh=_mesh,
    scratch_types=[
        pltpu.VMEM((4, R), jnp.int32),
        pltpu.VMEM((4, R), jnp.int32),
        pltpu.VMEM((R, H), jnp.float32),
        pltpu.VMEM((R, H), jnp.float32),
        pltpu.VMEM((R, H), jnp.float32),
        pltpu.VMEM((R, H), jnp.float32),
        pltpu.VMEM((R, H), jnp.float32),
        pltpu.VMEM((R, H), jnp.float32),
        pltpu.VMEM((R, H), jnp.float32),
        pltpu.VMEM((R, H), jnp.float32),
        pltpu.VMEM((R, 2 * H), jnp.float32),
        pltpu.VMEM_SHARED((NP, H), jnp.float32),
        pltpu.VMEM_SHARED((NP, H), jnp.float32),
    ] + [pltpu.SemaphoreType.DMA] * 8,
    compiler_params=pltpu.CompilerParams(use_tc_tiling_on_sc=False),
)
def _gather_kernel(idx4, a_hbm, b_hbm, g_hbm, ixA_v, ixB_v, a0, b0, c0, d0,
                   a1, b1, c1, d1, bufg_v, a_s, b_s, sa0, sb0, sc0, sd0,
                   sa1, sb1, sc1, sd1):
    cid = lax.axis_index("c")
    sid = lax.axis_index("s")
    wid = sid * NC + cid
    for t in range(RWP // R):
        sl = pl.ds(sid * RWP + t * R, R)
        pltpu.sync_copy(a_hbm.at[sl], a0)
        pltpu.sync_copy(a0, a_s.at[sl])
        pltpu.sync_copy(b_hbm.at[sl], b0)
        pltpu.sync_copy(b0, b_s.at[sl])
    plsc.subcore_barrier()

    def fire(ix_v, bufs, sems, chunk):
        pltpu.sync_copy(idx4.at[wid, chunk], ix_v)
        pltpu.async_copy(a_s.at[ix_v.at[0]], bufs[0], sems[0])
        pltpu.async_copy(b_s.at[ix_v.at[1]], bufs[1], sems[1])
        pltpu.async_copy(a_s.at[ix_v.at[2]], bufs[2], sems[2])
        pltpu.async_copy(b_s.at[ix_v.at[3]], bufs[3], sems[3])

    def drain(bufs, sems):
        for b, s in zip(bufs, sems):
            pltpu.make_async_copy(a_hbm.at[pl.ds(0, R)], b, s).wait()

    def adds_and_write(bufs, chunk):
        ba, bb, bc, bd = bufs

        def addrow8(i8, _):
            for di in range(8):
                i = i8 * 8 + di
                for k in range(H // 16):
                    sl = pl.ds(k * 16, 16)
                    bufg_v[i, sl] = ba[i, sl] + bb[i, sl]
                    bufg_v[i, pl.ds(H + k * 16, 16)] = bc[i, sl] + bd[i, sl]
            return 0

        lax.fori_loop(0, R // 8, addrow8, 0)
        pltpu.sync_copy(bufg_v,
                        g_hbm.at[pl.ds((wid * NCH3 + chunk) * R, R)])

    setA = ((a0, b0, c0, d0), (sa0, sb0, sc0, sd0))
    setB = ((a1, b1, c1, d1), (sa1, sb1, sc1, sd1))
    fire(ixA_v, setA[0], setA[1], 0)
    fire(ixB_v, setB[0], setB[1], 1)

    def body(i2, _):
        j = 2 * i2
        drain(setA[0], setA[1])
        adds_and_write(setA[0], j)
        fire(ixA_v, setA[0], setA[1], (j + 2) % NCH3)
        drain(setB[0], setB[1])
        adds_and_write(setB[0], j + 1)
        fire(ixB_v, setB[0], setB[1], (j + 3) % NCH3)
        return 0

    lax.fori_loop(0, NCH3 // 2, body, 0)
    drain(setA[0], setA[1])
    drain(setB[0], setB[1])


def _node1_body(x_ref, w1_ref, b1_ref, wg_ref, degc_ref, u_ref):
    h = jnp.maximum(
        jnp.dot(x_ref[...], w1_ref[...], preferred_element_type=jnp.float32)
        + b1_ref[...], 0.0)
    hw = jnp.dot(h, wg_ref[...], preferred_element_type=jnp.float32)
    d = degc_ref[...]
    dinv = lax.rsqrt(d[:, 0:1] + d[:, 1:2] + 1.0)
    u_ref[:N] = hw * dinv[:N, :]
    u_ref[N:] = jnp.zeros((NP - N, H), jnp.float32)


def _node2_body(sp_ref, u_ref, degc_ref, bg_ref, w3a_ref, w3b_ref, b3_ref,
                a_ref, b_ref):
    s = sp_ref[0] + sp_ref[1] + u_ref[...]
    d = degc_ref[...]
    dinv = lax.rsqrt(d[:, 0:1] + d[:, 1:2] + 1.0)
    agg = s * dinv + bg_ref[...]
    a_ref[...] = jnp.dot(agg, w3a_ref[...], preferred_element_type=jnp.float32)
    b_ref[...] = jnp.dot(agg, w3b_ref[...],
                         preferred_element_type=jnp.float32) + b3_ref[...]


BLK2 = 3200


def _edge_body(g_ref, eatl_ref, eatr_ref, w2_ref, b2_ref, w3c_ref, w4p_ref,
               b4_ref, o_ref):
    ea = jnp.concatenate([eatl_ref[...].T, eatr_ref[...].T], axis=1)
    ef2 = jnp.maximum(
        jnp.dot(ea, w2_ref[...], preferred_element_type=jnp.float32)
        + b2_ref[...], 0.0)
    efc2 = jnp.dot(ef2, w3c_ref[...], preferred_element_type=jnp.float32)
    hid2 = jnp.maximum(g_ref[...] + efc2, 0.0)
    o = jnp.dot(hid2, w4p_ref[...],
                preferred_element_type=jnp.float32) + b4_ref[...]
    o_ref[...] = o.T


def kernel(x, edge_index, edge_attr, W1, b1, W2, b2, Wg, bg, W3, b3, W4, b4):
    src_all = edge_index[0].astype(jnp.int32)
    dst_all = edge_index[1].astype(jnp.int32)
    src2 = src_all.reshape(NW, NCH, C)
    dst2 = dst_all.reshape(NW, NCH, C)
    srcc = src_all.reshape(NW, NCH2, C2)
    dstc = dst_all.reshape(NW, NCH2, C2)
    pad = (jnp.arange(E2P - E2, dtype=jnp.int32) * 131) % N
    sl2 = jnp.concatenate([src_all[:E2], pad]).reshape(NW, NCH3, R)
    dl2 = jnp.concatenate([dst_all[:E2], pad]).reshape(NW, NCH3, R)
    sr2 = jnp.concatenate([src_all[E2:], pad]).reshape(NW, NCH3, R)
    dr2 = jnp.concatenate([dst_all[E2:], pad]).reshape(NW, NCH3, R)
    idx4 = jnp.stack([sl2, dl2, sr2, dr2], axis=2)

    degp = _deg_kernel(dst2)
    degc = degp.T

    u = pl.pallas_call(
        _node1_body,
        out_shape=jax.ShapeDtypeStruct((NP, H), jnp.float32),
    )(x, W1, b1.reshape(1, H), Wg, degc)

    sp = _scatter_kernel(srcc, dstc, u)

    a_nodes, b_nodes = pl.pallas_call(
        _node2_body,
        out_shape=[
            jax.ShapeDtypeStruct((NP, H), jnp.float32),
            jax.ShapeDtypeStruct((NP, H), jnp.float32),
        ],
    )(sp, u, degc, bg.reshape(1, H), W3[:H], W3[H:2 * H], b3.reshape(1, H))

    g2 = _gather_kernel(idx4, a_nodes, b_nodes)
    eat = edge_attr.T
    z = jnp.zeros((H, 1), jnp.float32)
    w4p = jnp.concatenate([jnp.concatenate([W4, z], axis=1),
                           jnp.concatenate([z, W4], axis=1)], axis=0)
    zde = jnp.zeros((DE, H), jnp.float32)
    w2p = jnp.concatenate([jnp.concatenate([W2, zde], axis=1),
                           jnp.concatenate([zde, W2], axis=1)], axis=0)
    b2p = jnp.concatenate([b2, b2]).reshape(1, 2 * H)
    zh = jnp.zeros((H, H), jnp.float32)
    w3c = W3[2 * H:]
    w3cp = jnp.concatenate([jnp.concatenate([w3c, zh], axis=1),
                            jnp.concatenate([zh, w3c], axis=1)], axis=0)

    out = pl.pallas_call(
        _edge_body,
        grid=(E2 // BLK2,),
        in_specs=[
            pl.BlockSpec((BLK2, 2 * H), lambda i: (i, 0)),
            pl.BlockSpec((DE, BLK2), lambda i: (0, i)),
            pl.BlockSpec((DE, BLK2), lambda i: (0, i + E2 // BLK2)),
            pl.BlockSpec((2 * DE, 2 * H), lambda i: (0, 0)),
            pl.BlockSpec((1, 2 * H), lambda i: (0, 0)),
            pl.BlockSpec((2 * H, 2 * H), lambda i: (0, 0)),
            pl.BlockSpec((2 * H, 2), lambda i: (0, 0)),
            pl.BlockSpec((1, 1), lambda i: (0, 0)),
        ],
        out_specs=pl.BlockSpec((2, BLK2), lambda i: (0, i)),
        out_shape=jax.ShapeDtypeStruct((2, E2), jnp.float32),
    )(g2, eat, eat, w2p, b2p, w3cp, w4p, b4.reshape(1, 1))

    return out.reshape(-1)
